# Initial kernel scaffold; baseline (speedup 1.0000x reference)
import jax
import jax.numpy as jnp
from jax.experimental import pallas as pl


def kernel(query, key, value):
    raise NotImplementedError("write your pallas kernel here")


# speedup vs baseline: 2.3581x; 2.3581x over previous
import jax
import jax.numpy as jnp
from jax.experimental import pallas as pl
from jax.experimental.pallas import tpu as pltpu

N_HEADS = 16
N_KV_HEADS = 4
HEAD_DIM = 128
ATTN_SCALE = 0.08838834764831845

BQ = 512
BK = 512

NEG_INF = float("-inf")


def _flash_body(q_ref, k_ref, v_ref, o_ref):
    i = pl.program_id(1)
    group = N_HEADS // N_KV_HEADS

    rows = jax.lax.broadcasted_iota(jnp.int32, (BQ, BK), 0)
    cols = jax.lax.broadcasted_iota(jnp.int32, (BQ, BK), 1)
    diag_mask = cols <= rows

    for h in range(N_HEADS):
        kvh = h // group
        qs = h * HEAD_DIM
        ks = kvh * HEAD_DIM
        q = (q_ref[0, :, qs:qs + HEAD_DIM] * ATTN_SCALE).astype(jnp.bfloat16)

        acc0 = jnp.zeros((BQ, HEAD_DIM), jnp.float32)
        m0 = jnp.full((BQ, 1), NEG_INF, jnp.float32)
        l0 = jnp.zeros((BQ, 1), jnp.float32)

        def inner(j, carry, q=q, ks=ks):
            acc, m, l = carry
            kb = k_ref[0, pl.ds(j * BK, BK), ks:ks + HEAD_DIM].astype(
                jnp.bfloat16)
            s = jax.lax.dot_general(q, kb, (((1,), (1,)), ((), ())),
                                    preferred_element_type=jnp.float32)
            m_new = jnp.maximum(m, jnp.max(s, axis=1, keepdims=True))
            p = jnp.exp(s - m_new)
            alpha = jnp.exp(m - m_new)
            l = l * alpha + jnp.sum(p, axis=1, keepdims=True)
            vb = v_ref[0, pl.ds(j * BK, BK), ks:ks + HEAD_DIM].astype(
                jnp.bfloat16)
            pv = jax.lax.dot_general(p.astype(jnp.bfloat16), vb,
                                     (((1,), (0,)), ((), ())),
                                     preferred_element_type=jnp.float32)
            acc = acc * alpha + pv
            return acc, m_new, l

        acc, m, l = jax.lax.fori_loop(0, i, inner, (acc0, m0, l0))

        kb = k_ref[0, pl.ds(i * BK, BK), ks:ks + HEAD_DIM].astype(jnp.bfloat16)
        s = jax.lax.dot_general(q, kb, (((1,), (1,)), ((), ())),
                                preferred_element_type=jnp.float32)
        s = jnp.where(diag_mask, s, NEG_INF)
        m_new = jnp.maximum(m, jnp.max(s, axis=1, keepdims=True))
        p = jnp.exp(s - m_new)
        alpha = jnp.exp(m - m_new)
        l = l * alpha + jnp.sum(p, axis=1, keepdims=True)
        vb = v_ref[0, pl.ds(i * BK, BK), ks:ks + HEAD_DIM].astype(jnp.bfloat16)
        pv = jax.lax.dot_general(p.astype(jnp.bfloat16), vb,
                                 (((1,), (0,)), ((), ())),
                                 preferred_element_type=jnp.float32)
        acc = acc * alpha + pv

        o_ref[0, :, qs:qs + HEAD_DIM] = (acc / l).astype(jnp.float32)


def kernel(query, key, value):
    B, S, QF = query.shape
    KF = key.shape[-1]

    return pl.pallas_call(
        _flash_body,
        grid=(B, S // BQ),
        in_specs=[
            pl.BlockSpec((1, BQ, QF), lambda b, i: (b, i, 0)),
            pl.BlockSpec((1, S, KF), lambda b, i: (b, 0, 0)),
            pl.BlockSpec((1, S, KF), lambda b, i: (b, 0, 0)),
        ],
        out_specs=pl.BlockSpec((1, BQ, QF), lambda b, i: (b, i, 0)),
        out_shape=jax.ShapeDtypeStruct((B, S, QF), jnp.float32),
        compiler_params=pltpu.CompilerParams(
            dimension_semantics=("parallel", "arbitrary")),
    )(query, key, value)
